# baseline (speedup 1.0000x reference)
import jax
import jax.numpy as jnp
from jax.experimental import pallas as pl
import jax.experimental.pallas.tpu as pltpu

B = 64
C = 256
D = C * 4
NUM_EXPERTS = 4
NUM_CLASSES = 6

HW0 = 56 * 56
HW1 = 28 * 28
HW2 = 14 * 14
HW3 = 7 * 7

NBUF0 = 4
NBUF1 = 2
NBUF2 = 2

R0 = C
R1 = 4 * C
R2 = 16 * C
N0 = B
N1 = B // 4
N2 = B // 16


def _pool_body(f0, f1, f2, f3, out,
               buf0, buf1, buf2, buf3, sem0, sem1, sem2, sem3):
    def cp0(k, slot):
        return pltpu.make_async_copy(
            f0.at[pl.ds(k * R0, R0), :], buf0.at[slot], sem0.at[slot])

    def cp1(k, slot):
        return pltpu.make_async_copy(
            f1.at[pl.ds(k * R1, R1), :], buf1.at[slot], sem1.at[slot])

    def cp2(k, slot):
        return pltpu.make_async_copy(
            f2.at[pl.ds(k * R2, R2), :], buf2.at[slot], sem2.at[slot])

    pltpu.make_async_copy(f3, buf3, sem3).start()
    for k in range(NBUF2):
        cp2(k, k).start()
    for k in range(NBUF1):
        cp1(k, k).start()
    for k in range(NBUF0):
        cp0(k, k).start()

    def body0(k, carry):
        slot = jax.lax.rem(k, NBUF0)
        cp0(k, slot).wait()
        out[k, 0, :] = jnp.sum(buf0[slot], axis=1) * (1.0 / HW0)

        @pl.when(k + NBUF0 < N0)
        def _():
            cp0(k + NBUF0, jax.lax.rem(k + NBUF0, NBUF0)).start()
        return carry

    jax.lax.fori_loop(0, N0, body0, 0)

    def body1(k, carry):
        slot = jax.lax.rem(k, NBUF1)
        cp1(k, slot).wait()
        s = jnp.sum(buf1[slot], axis=1) * (1.0 / HW1)
        for i in range(4):
            out[4 * k + i, 1, :] = s[i * C:(i + 1) * C]

        @pl.when(k + NBUF1 < N1)
        def _():
            cp1(k + NBUF1, jax.lax.rem(k + NBUF1, NBUF1)).start()
        return carry

    jax.lax.fori_loop(0, N1, body1, 0)

    def body2(k, carry):
        slot = jax.lax.rem(k, NBUF2)
        cp2(k, slot).wait()
        s = jnp.sum(buf2[slot], axis=1) * (1.0 / HW2)
        for i in range(16):
            out[16 * k + i, 2, :] = s[i * C:(i + 1) * C]

        @pl.when(k + NBUF2 < N2)
        def _():
            cp2(k + NBUF2, jax.lax.rem(k + NBUF2, NBUF2)).start()
        return carry

    jax.lax.fori_loop(0, N2, body2, 0)

    pltpu.make_async_copy(f3, buf3, sem3).wait()
    s3 = jnp.sum(buf3[...], axis=1) * (1.0 / HW3)
    for b in range(B):
        out[b, 3, :] = s3[b * C:(b + 1) * C]


def _head_body(pooled, wg, bg, we, be, out, gw_out):
    feat = pooled[...]
    gate = jax.lax.dot_general(
        feat, wg[...], (((1,), (0,)), ((), ())),
        preferred_element_type=jnp.float32) + bg[...]
    m = jnp.max(gate, axis=1, keepdims=True)
    ex = jnp.exp(gate - m)
    gw = ex / jnp.sum(ex, axis=1, keepdims=True)
    acc = jnp.zeros((B, NUM_CLASSES), dtype=jnp.float32)
    for e in range(NUM_EXPERTS):
        eo = jax.lax.dot_general(
            feat, we[e], (((1,), (0,)), ((), ())),
            preferred_element_type=jnp.float32) + be[e:e + 1, :]
        acc = acc + gw[:, e:e + 1] * eo
    out[...] = acc
    gw_out[...] = gw


def kernel(feature_0, feature_1, feature_2, feature_3, c_feature, t_feature,
           Wg, bg, We, be):
    del c_feature, t_feature
    f0 = feature_0.reshape(B * C, HW0)
    f1 = feature_1.reshape(B * C, HW1)
    f2 = feature_2.reshape(B * C, HW2)
    f3 = feature_3.reshape(B * C, HW3)
    pooled = pl.pallas_call(
        _pool_body,
        in_specs=[
            pl.BlockSpec(memory_space=pltpu.HBM),
            pl.BlockSpec(memory_space=pltpu.HBM),
            pl.BlockSpec(memory_space=pltpu.HBM),
            pl.BlockSpec(memory_space=pltpu.HBM),
        ],
        out_specs=pl.BlockSpec(memory_space=pltpu.VMEM),
        out_shape=jax.ShapeDtypeStruct((B, NUM_EXPERTS, C), jnp.float32),
        scratch_shapes=[
            pltpu.VMEM((NBUF0, R0, HW0), jnp.float32),
            pltpu.VMEM((NBUF1, R1, HW1), jnp.float32),
            pltpu.VMEM((NBUF2, R2, HW2), jnp.float32),
            pltpu.VMEM((B * C, HW3), jnp.float32),
            pltpu.SemaphoreType.DMA((NBUF0,)),
            pltpu.SemaphoreType.DMA((NBUF1,)),
            pltpu.SemaphoreType.DMA((NBUF2,)),
            pltpu.SemaphoreType.DMA,
        ],
    )(f0, f1, f2, f3)

    feat = pooled.reshape(B, D)
    out, gw = pl.pallas_call(
        _head_body,
        in_specs=[
            pl.BlockSpec((B, D), lambda: (0, 0)),
            pl.BlockSpec(Wg.shape, lambda: (0, 0)),
            pl.BlockSpec((1, NUM_EXPERTS), lambda: (0, 0)),
            pl.BlockSpec(We.shape, lambda: (0, 0, 0)),
            pl.BlockSpec(be.shape, lambda: (0, 0)),
        ],
        out_specs=[
            pl.BlockSpec((B, NUM_CLASSES), lambda: (0, 0)),
            pl.BlockSpec((B, NUM_EXPERTS), lambda: (0, 0)),
        ],
        out_shape=[
            jax.ShapeDtypeStruct((B, NUM_CLASSES), jnp.float32),
            jax.ShapeDtypeStruct((B, NUM_EXPERTS), jnp.float32),
        ],
    )(feat, Wg, bg.reshape(1, NUM_EXPERTS), We, be)
    return (out, gw)


# speedup vs baseline: 1.9429x; 1.9429x over previous
import jax
import jax.numpy as jnp
from jax.experimental import pallas as pl
import jax.experimental.pallas.tpu as pltpu

B = 64
C = 256
D = C * 4
NUM_EXPERTS = 4
NUM_CLASSES = 6

HW0 = 56 * 56
HW1 = 28 * 28
HW2 = 14 * 14
HW3 = 7 * 7

NBUF0 = 4
NBUF1 = 2
NBUF2 = 2

G1 = 4
G2 = 16
N1 = B // G1
N2 = B // G2


def _pool_body(f0, f1, f2, f3, out,
               buf0, buf1, buf2, buf3, sem0, sem1, sem2, sem3):
    def cp0(k, slot):
        return pltpu.make_async_copy(f0.at[k], buf0.at[slot], sem0.at[slot])

    def cp1(k, slot):
        return pltpu.make_async_copy(
            f1.at[pl.ds(k * G1, G1)], buf1.at[slot], sem1.at[slot])

    def cp2(k, slot):
        return pltpu.make_async_copy(
            f2.at[pl.ds(k * G2, G2)], buf2.at[slot], sem2.at[slot])

    pltpu.make_async_copy(f3, buf3, sem3).start()
    for k in range(NBUF2):
        cp2(k, k).start()
    for k in range(NBUF1):
        cp1(k, k).start()
    for k in range(NBUF0):
        cp0(k, k).start()

    def body0(k, carry):
        slot = jax.lax.rem(k, NBUF0)
        cp0(k, slot).wait()
        out[k, 0, :] = jnp.sum(buf0[slot], axis=1) * (1.0 / HW0)

        @pl.when(k + NBUF0 < B)
        def _():
            cp0(k + NBUF0, jax.lax.rem(k + NBUF0, NBUF0)).start()
        return carry

    jax.lax.fori_loop(0, B, body0, 0)

    def body1(k, carry):
        slot = jax.lax.rem(k, NBUF1)
        cp1(k, slot).wait()
        out[pl.ds(k * G1, G1), 1, :] = (
            jnp.sum(buf1[slot], axis=2) * (1.0 / HW1))

        @pl.when(k + NBUF1 < N1)
        def _():
            cp1(k + NBUF1, jax.lax.rem(k + NBUF1, NBUF1)).start()
        return carry

    jax.lax.fori_loop(0, N1, body1, 0)

    def body2(k, carry):
        slot = jax.lax.rem(k, NBUF2)
        cp2(k, slot).wait()
        out[pl.ds(k * G2, G2), 2, :] = (
            jnp.sum(buf2[slot], axis=2) * (1.0 / HW2))

        @pl.when(k + NBUF2 < N2)
        def _():
            cp2(k + NBUF2, jax.lax.rem(k + NBUF2, NBUF2)).start()
        return carry

    jax.lax.fori_loop(0, N2, body2, 0)

    pltpu.make_async_copy(f3, buf3, sem3).wait()
    out[:, 3, :] = jnp.sum(buf3[...], axis=2) * (1.0 / HW3)


def _head_body(pooled, wg, bg, we, be, out, gw_out):
    feat = pooled[...]
    gate = jax.lax.dot_general(
        feat, wg[...], (((1,), (0,)), ((), ())),
        preferred_element_type=jnp.float32) + bg[...]
    m = jnp.max(gate, axis=1, keepdims=True)
    ex = jnp.exp(gate - m)
    gw = ex / jnp.sum(ex, axis=1, keepdims=True)
    acc = jnp.zeros((B, NUM_CLASSES), dtype=jnp.float32)
    for e in range(NUM_EXPERTS):
        eo = jax.lax.dot_general(
            feat, we[e], (((1,), (0,)), ((), ())),
            preferred_element_type=jnp.float32) + be[e:e + 1, :]
        acc = acc + gw[:, e:e + 1] * eo
    out[...] = acc
    gw_out[...] = gw


def kernel(feature_0, feature_1, feature_2, feature_3, c_feature, t_feature,
           Wg, bg, We, be):
    del c_feature, t_feature
    f0 = feature_0.reshape(B, C, HW0)
    f1 = feature_1.reshape(B, C, HW1)
    f2 = feature_2.reshape(B, C, HW2)
    f3 = feature_3.reshape(B, C, HW3)
    pooled = pl.pallas_call(
        _pool_body,
        in_specs=[
            pl.BlockSpec(memory_space=pltpu.HBM),
            pl.BlockSpec(memory_space=pltpu.HBM),
            pl.BlockSpec(memory_space=pltpu.HBM),
            pl.BlockSpec(memory_space=pltpu.HBM),
        ],
        out_specs=pl.BlockSpec(memory_space=pltpu.VMEM),
        out_shape=jax.ShapeDtypeStruct((B, NUM_EXPERTS, C), jnp.float32),
        scratch_shapes=[
            pltpu.VMEM((NBUF0, C, HW0), jnp.float32),
            pltpu.VMEM((NBUF1, G1, C, HW1), jnp.float32),
            pltpu.VMEM((NBUF2, G2, C, HW2), jnp.float32),
            pltpu.VMEM((B, C, HW3), jnp.float32),
            pltpu.SemaphoreType.DMA((NBUF0,)),
            pltpu.SemaphoreType.DMA((NBUF1,)),
            pltpu.SemaphoreType.DMA((NBUF2,)),
            pltpu.SemaphoreType.DMA,
        ],
    )(f0, f1, f2, f3)

    feat = pooled.reshape(B, D)
    out, gw = pl.pallas_call(
        _head_body,
        in_specs=[
            pl.BlockSpec((B, D), lambda: (0, 0)),
            pl.BlockSpec(Wg.shape, lambda: (0, 0)),
            pl.BlockSpec((1, NUM_EXPERTS), lambda: (0, 0)),
            pl.BlockSpec(We.shape, lambda: (0, 0, 0)),
            pl.BlockSpec(be.shape, lambda: (0, 0)),
        ],
        out_specs=[
            pl.BlockSpec((B, NUM_CLASSES), lambda: (0, 0)),
            pl.BlockSpec((B, NUM_EXPERTS), lambda: (0, 0)),
        ],
        out_shape=[
            jax.ShapeDtypeStruct((B, NUM_CLASSES), jnp.float32),
            jax.ShapeDtypeStruct((B, NUM_EXPERTS), jnp.float32),
        ],
    )(feat, Wg, bg.reshape(1, NUM_EXPERTS), We, be)
    return (out, gw)


# speedup vs baseline: 1.9535x; 1.0055x over previous
import jax
import jax.numpy as jnp
from jax.experimental import pallas as pl
import jax.experimental.pallas.tpu as pltpu

B = 64
C = 256
D = C * 4
NUM_EXPERTS = 4
NUM_CLASSES = 6

HW0 = 56 * 56
HW1 = 28 * 28
HW2 = 14 * 14
HW3 = 7 * 7

WAY0 = 4
NBUF0 = 2 * WAY0
NSTEP0 = B // WAY0
NBUF1 = 2
NBUF2 = 2
NBUF3 = 2

G1 = 4
G2 = 8
G3 = 16
N1 = B // G1
N2 = B // G2
N3 = B // G3


def _pool_body(f0, f1, f2, f3, out,
               buf0, buf1, buf2, buf3, sem0, sem1, sem2, sem3):
    def cp0(k, slot):
        return pltpu.make_async_copy(f0.at[k], buf0.at[slot], sem0.at[slot])

    def cp1(k, slot):
        return pltpu.make_async_copy(
            f1.at[pl.ds(k * G1, G1)], buf1.at[slot], sem1.at[slot])

    def cp2(k, slot):
        return pltpu.make_async_copy(
            f2.at[pl.ds(k * G2, G2)], buf2.at[slot], sem2.at[slot])

    def cp3(k, slot):
        return pltpu.make_async_copy(
            f3.at[pl.ds(k * G3, G3)], buf3.at[slot], sem3.at[slot])

    for k in range(NBUF3):
        cp3(k, k).start(priority=1)
    for k in range(NBUF2):
        cp2(k, k).start(priority=0)
    for k in range(NBUF1):
        cp1(k, k).start(priority=k % 2)
    for k in range(NBUF0):
        cp0(k, k).start(priority=k % 2)

    def body0(j, carry):
        base = j * WAY0
        for i in range(WAY0):
            k = base + i
            slot = jax.lax.rem(k, NBUF0)
            cp0(k, slot).wait()
            out[k, 0, :] = jnp.sum(buf0[slot], axis=1) * (1.0 / HW0)

            @pl.when(k + NBUF0 < B)
            def _():
                cp0(k + NBUF0, slot).start(priority=i % 2)
        return carry

    jax.lax.fori_loop(0, NSTEP0, body0, 0)

    def body1(k, carry):
        slot = jax.lax.rem(k, NBUF1)
        cp1(k, slot).wait()
        out[pl.ds(k * G1, G1), 1, :] = (
            jnp.sum(buf1[slot], axis=2) * (1.0 / HW1))

        @pl.when(k + NBUF1 < N1)
        def _():
            cp1(k + NBUF1, slot).start(priority=1)
        return carry

    jax.lax.fori_loop(0, N1, body1, 0)

    def body2(k, carry):
        slot = jax.lax.rem(k, NBUF2)
        cp2(k, slot).wait()
        out[pl.ds(k * G2, G2), 2, :] = (
            jnp.sum(buf2[slot], axis=2) * (1.0 / HW2))

        @pl.when(k + NBUF2 < N2)
        def _():
            cp2(k + NBUF2, slot).start(priority=0)
        return carry

    jax.lax.fori_loop(0, N2, body2, 0)

    def body3(k, carry):
        slot = jax.lax.rem(k, NBUF3)
        cp3(k, slot).wait()
        out[pl.ds(k * G3, G3), 3, :] = (
            jnp.sum(buf3[slot], axis=2) * (1.0 / HW3))

        @pl.when(k + NBUF3 < N3)
        def _():
            cp3(k + NBUF3, slot).start(priority=1)
        return carry

    jax.lax.fori_loop(0, N3, body3, 0)


def _head_body(pooled, wg, bg, we, be, out, gw_out):
    feat = pooled[...]
    gate = jax.lax.dot_general(
        feat, wg[...], (((1,), (0,)), ((), ())),
        preferred_element_type=jnp.float32) + bg[...]
    m = jnp.max(gate, axis=1, keepdims=True)
    ex = jnp.exp(gate - m)
    gw = ex / jnp.sum(ex, axis=1, keepdims=True)
    acc = jnp.zeros((B, NUM_CLASSES), dtype=jnp.float32)
    for e in range(NUM_EXPERTS):
        eo = jax.lax.dot_general(
            feat, we[e], (((1,), (0,)), ((), ())),
            preferred_element_type=jnp.float32) + be[e:e + 1, :]
        acc = acc + gw[:, e:e + 1] * eo
    out[...] = acc
    gw_out[...] = gw


def kernel(feature_0, feature_1, feature_2, feature_3, c_feature, t_feature,
           Wg, bg, We, be):
    del c_feature, t_feature
    f0 = feature_0.reshape(B, C, HW0)
    f1 = feature_1.reshape(B, C, HW1)
    f2 = feature_2.reshape(B, C, HW2)
    f3 = feature_3.reshape(B, C, HW3)
    pooled = pl.pallas_call(
        _pool_body,
        in_specs=[
            pl.BlockSpec(memory_space=pltpu.HBM),
            pl.BlockSpec(memory_space=pltpu.HBM),
            pl.BlockSpec(memory_space=pltpu.HBM),
            pl.BlockSpec(memory_space=pltpu.HBM),
        ],
        out_specs=pl.BlockSpec(memory_space=pltpu.VMEM),
        out_shape=jax.ShapeDtypeStruct((B, NUM_EXPERTS, C), jnp.float32),
        scratch_shapes=[
            pltpu.VMEM((NBUF0, C, HW0), jnp.float32),
            pltpu.VMEM((NBUF1, G1, C, HW1), jnp.float32),
            pltpu.VMEM((NBUF2, G2, C, HW2), jnp.float32),
            pltpu.VMEM((NBUF3, G3, C, HW3), jnp.float32),
            pltpu.SemaphoreType.DMA((NBUF0,)),
            pltpu.SemaphoreType.DMA((NBUF1,)),
            pltpu.SemaphoreType.DMA((NBUF2,)),
            pltpu.SemaphoreType.DMA((NBUF3,)),
        ],
    )(f0, f1, f2, f3)

    feat = pooled.reshape(B, D)
    out, gw = pl.pallas_call(
        _head_body,
        in_specs=[
            pl.BlockSpec((B, D), lambda: (0, 0)),
            pl.BlockSpec(Wg.shape, lambda: (0, 0)),
            pl.BlockSpec((1, NUM_EXPERTS), lambda: (0, 0)),
            pl.BlockSpec(We.shape, lambda: (0, 0, 0)),
            pl.BlockSpec(be.shape, lambda: (0, 0)),
        ],
        out_specs=[
            pl.BlockSpec((B, NUM_CLASSES), lambda: (0, 0)),
            pl.BlockSpec((B, NUM_EXPERTS), lambda: (0, 0)),
        ],
        out_shape=[
            jax.ShapeDtypeStruct((B, NUM_CLASSES), jnp.float32),
            jax.ShapeDtypeStruct((B, NUM_EXPERTS), jnp.float32),
        ],
    )(feat, Wg, bg.reshape(1, NUM_EXPERTS), We, be)
    return (out, gw)


# speedup vs baseline: 2.0222x; 1.0352x over previous
import jax
import jax.numpy as jnp
from jax.experimental import pallas as pl
import jax.experimental.pallas.tpu as pltpu

B = 64
C = 256
D = C * 4
NUM_EXPERTS = 4
NUM_CLASSES = 6

C_CHUNK = 128


BB = 1


def _pool_body(f0a, f0b, f0c, f0d, f1a, f1b, f2, f3, out):
    for i in range(BB):
        out[i, 0, 0:64] = jnp.sum(f0a[i], axis=1) * (1.0 / (56 * 56))
        out[i, 0, 64:128] = jnp.sum(f0b[i], axis=1) * (1.0 / (56 * 56))
        out[i, 0, 128:192] = jnp.sum(f0c[i], axis=1) * (1.0 / (56 * 56))
        out[i, 0, 192:256] = jnp.sum(f0d[i], axis=1) * (1.0 / (56 * 56))
        out[i, 1, 0:128] = jnp.sum(f1a[i], axis=1) * (1.0 / (28 * 28))
        out[i, 1, 128:256] = jnp.sum(f1b[i], axis=1) * (1.0 / (28 * 28))
        out[i, 2, :] = jnp.sum(f2[i], axis=1) * (1.0 / (14 * 14))
        out[i, 3, :] = jnp.sum(f3[i], axis=1) * (1.0 / (7 * 7))


def _head_body(pooled, wg, bg, we, be, out, gw_out):
    feat = pooled[...]
    gate = jax.lax.dot_general(
        feat, wg[...], (((1,), (0,)), ((), ())),
        preferred_element_type=jnp.float32) + bg[...]
    m = jnp.max(gate, axis=1, keepdims=True)
    ex = jnp.exp(gate - m)
    gw = ex / jnp.sum(ex, axis=1, keepdims=True)
    acc = jnp.zeros((B, NUM_CLASSES), dtype=jnp.float32)
    for e in range(NUM_EXPERTS):
        eo = jax.lax.dot_general(
            feat, we[e], (((1,), (0,)), ((), ())),
            preferred_element_type=jnp.float32) + be[e:e + 1, :]
        acc = acc + gw[:, e:e + 1] * eo
    out[...] = acc
    gw_out[...] = gw


def kernel(feature_0, feature_1, feature_2, feature_3, c_feature, t_feature,
           Wg, bg, We, be):
    del c_feature, t_feature
    f0 = feature_0.reshape(B, C, 56 * 56)
    f1 = feature_1.reshape(B, C, 28 * 28)
    f2 = feature_2.reshape(B, C, 14 * 14)
    f3 = feature_3.reshape(B, C, 7 * 7)
    pooled = pl.pallas_call(
        _pool_body,
        grid=(B // BB,),
        in_specs=[
            pl.BlockSpec((BB, 64, 56 * 56), lambda b: (b, 0, 0)),
            pl.BlockSpec((BB, 64, 56 * 56), lambda b: (b, 1, 0)),
            pl.BlockSpec((BB, 64, 56 * 56), lambda b: (b, 2, 0)),
            pl.BlockSpec((BB, 64, 56 * 56), lambda b: (b, 3, 0)),
            pl.BlockSpec((BB, 128, 28 * 28), lambda b: (b, 0, 0)),
            pl.BlockSpec((BB, 128, 28 * 28), lambda b: (b, 1, 0)),
            pl.BlockSpec((BB, C, 14 * 14), lambda b: (b, 0, 0)),
            pl.BlockSpec((BB, C, 7 * 7), lambda b: (b, 0, 0)),
        ],
        out_specs=pl.BlockSpec((BB, NUM_EXPERTS, C), lambda b: (b, 0, 0)),
        out_shape=jax.ShapeDtypeStruct((B, NUM_EXPERTS, C), jnp.float32),
        compiler_params=pltpu.CompilerParams(
            dimension_semantics=("arbitrary",),
            allow_input_fusion=[True, True, True, True, True, True, True, True]),
    )(f0, f0, f0, f0, f1, f1, f2, f3)

    feat = pooled.reshape(B, D)
    out, gw = pl.pallas_call(
        _head_body,
        in_specs=[
            pl.BlockSpec((B, D), lambda: (0, 0)),
            pl.BlockSpec(Wg.shape, lambda: (0, 0)),
            pl.BlockSpec((1, NUM_EXPERTS), lambda: (0, 0)),
            pl.BlockSpec(We.shape, lambda: (0, 0, 0)),
            pl.BlockSpec(be.shape, lambda: (0, 0)),
        ],
        out_specs=[
            pl.BlockSpec((B, NUM_CLASSES), lambda: (0, 0)),
            pl.BlockSpec((B, NUM_EXPERTS), lambda: (0, 0)),
        ],
        out_shape=[
            jax.ShapeDtypeStruct((B, NUM_CLASSES), jnp.float32),
            jax.ShapeDtypeStruct((B, NUM_EXPERTS), jnp.float32),
        ],
    )(feat, Wg, bg.reshape(1, NUM_EXPERTS), We, be)
    return (out, gw)
